# baseline (speedup 1.0000x reference)
import functools

import jax
import jax.numpy as jnp
from jax import lax
from jax.experimental import pallas as pl
from jax.experimental.pallas import tpu as pltpu
from jax.experimental.pallas import tpu_sc as plsc

N = 10000
E = 320000
D = 128

NC = 2
NS = 16
NW = NC * NS
EPW = E // NW
CH = 80
KC = EPW // CH
RPS = 624
TAIL = N - NS * RPS

ER = 512
NR = 512


def _vs_mesh():
    return plsc.VectorSubcoreMesh(core_axis_name="c", subcore_axis_name="s",
                                  num_cores=NC, num_subcores=NS)


@functools.partial(
    pl.kernel,
    mesh=_vs_mesh(),
    out_type=(
        jax.ShapeDtypeStruct((NW, KC, CH, D), jnp.float32),
        jax.ShapeDtypeStruct((NW, KC, CH, D), jnp.float32),
    ),
    scratch_types=[
        pltpu.VMEM((KC, CH), jnp.int32),
        pltpu.VMEM((KC, CH), jnp.int32),
        pltpu.VMEM((CH, D), jnp.float32),
        pltpu.VMEM((CH, D), jnp.float32),
        pltpu.SemaphoreType.DMA,
        pltpu.SemaphoreType.DMA,
    ],
)
def _sc_gather(nf_hbm, src_hbm, dst_hbm, gs_hbm, gd_hbm,
               idxs, idxd, bufa, bufb, sema, semb):
    wid = lax.axis_index("s") * NC + lax.axis_index("c")
    pltpu.sync_copy(src_hbm.at[wid], idxs)
    pltpu.sync_copy(dst_hbm.at[wid], idxd)

    @pl.loop(0, KC)
    def _(c):
        ca = pltpu.async_copy(nf_hbm.at[idxs.at[c]], bufa, sema)
        cb = pltpu.async_copy(nf_hbm.at[idxd.at[c]], bufb, semb)
        ca.wait()
        cb.wait()
        pltpu.sync_copy(bufa, gs_hbm.at[wid, c])
        pltpu.sync_copy(bufb, gd_hbm.at[wid, c])


NRANGE = 312
ACCR = 336
DUMP = 328
SCH = 8000
CAPC = 156
CAPF = CAPC * CH


@functools.partial(
    pl.kernel,
    mesh=_vs_mesh(),
    out_type=jax.ShapeDtypeStruct((N, D), jnp.float32),
    scratch_types=[
        pltpu.VMEM_SHARED((NS, ACCR, D), jnp.float32),
        pltpu.VMEM((CAPC, CH), jnp.int32),
        pltpu.VMEM((CAPC, CH), jnp.int32),
        pltpu.VMEM((CH, D), jnp.float32),
        pltpu.VMEM((CH, D), jnp.float32),
        pltpu.SemaphoreType.DMA,
        pltpu.SemaphoreType.DMA,
    ],
)
def _sc_scatter(ef_hbm, eids_hbm, ldst_hbm, zer_hbm, out_hbm,
                acc, eids, ldst, vb0, vb1, sem0, sem1):
    cid = lax.axis_index("c")
    sid = lax.axis_index("s")
    wid = sid * NC + cid
    pltpu.sync_copy(zer_hbm, acc.at[sid])
    pltpu.sync_copy(eids_hbm.at[wid], eids)
    pltpu.sync_copy(ldst_hbm.at[wid], ldst)

    @pl.loop(0, CAPC)
    def _(c):
        pltpu.async_copy(ef_hbm.at[eids.at[c]], vb0, sem0).wait()
        pltpu.sync_copy(vb0, acc.at[sid].at[ldst.at[c]], add=True)

    pltpu.sync_copy(acc.at[sid].at[pl.ds(0, NRANGE)],
                    out_hbm.at[pl.ds(wid * NRANGE, NRANGE)])

    @pl.when(wid == NW - 1)
    def _():
        pltpu.sync_copy(acc.at[sid].at[pl.ds(NRANGE, TAIL)],
                        out_hbm.at[pl.ds(NW * NRANGE, TAIL)])


def _full(shape):
    return pl.BlockSpec(shape, lambda i: (0,) * len(shape))


def _rows(r, width):
    return pl.BlockSpec((r, width), lambda i: (i, 0))


def _mm(a, b):
    return jnp.dot(a, b, preferred_element_type=jnp.float32)


def _k_nf0(x, wne, bne, nf):
    nf[...] = _mm(x[...], wne[...]) + bne[...]


def _k_ef0(ea, wee, bee, ef):
    ef[...] = _mm(ea[...], wee[...]) + bee[...]


def _k_edge1(ef0, gs, gd, w1, b1, w2, b2, ef1):
    e_in = jnp.concatenate([gs[...], gd[...], ef0[...]], axis=1)
    h = jax.nn.relu(_mm(e_in, w1[...]) + b1[...])
    ef1[...] = jax.nn.relu(_mm(h, w2[...]) + b2[...])


def _k_edge2(ef1, gs, gd, w1, b1, w2, b2, weo, beo, wec, bec, ef2, pe):
    e_in = jnp.concatenate([gs[...], gd[...], ef1[...]], axis=1)
    h = jax.nn.relu(_mm(e_in, w1[...]) + b1[...])
    e2 = jax.nn.relu(_mm(h, w2[...]) + b2[...])
    ef2[...] = e2
    eo = jax.nn.relu(_mm(e2, weo[...]) + beo[...])
    pe[...] = _mm(eo, wec[...]) + bec[...]


def _k_edge3(ef2, gs, gd, weo, beo, w21, b21, w22, b22, efx, efy):
    e2 = ef2[...]
    eo = jax.nn.relu(_mm(e2, weo[...]) + beo[...])
    e_in = jnp.concatenate([gs[...], gd[...], eo, e2], axis=1)
    h = jax.nn.relu(_mm(e_in, w21[...]) + b21[...])
    e3 = jax.nn.relu(_mm(h, w22[...]) + b22[...])
    efx[...] = e3[:, :128]
    efy[...] = e3[:, 32:]


def _k_edge4(efx, efy, gs, gd, w21, b21, w22, b22, ox, oy):
    e_in = jnp.concatenate([gs[...], gd[...], efx[...], efy[...][:, 96:]],
                           axis=1)
    h = jax.nn.relu(_mm(e_in, w21[...]) + b21[...])
    e4 = jax.nn.relu(_mm(h, w22[...]) + b22[...])
    ox[...] = e4[:, :128]
    oy[...] = e4[:, 32:]


def _k_node(nf, p, wn, bn, nfo):
    cat = jnp.concatenate([nf[...], p[...]], axis=1)
    nfo[...] = jax.nn.relu(_mm(cat, wn[...]) + bn[...])


def _node2_cat(nf, px, py):
    return jnp.concatenate([nf[...], px[...], py[...][:, 96:]], axis=1)


def _k_node2(nf, px, py, wn, bn, nfo):
    cat = _node2_cat(nf, px, py)
    nfo[...] = jax.nn.relu(_mm(cat, wn[...]) + bn[...])


def _k_node_final(nf, px, py, wn, bn, wnc1, bnc1, wnc2, bnc2,
                  wc1, bc1, wc2, bc2, pn, pc):
    cat = _node2_cat(nf, px, py)
    nn = jax.nn.relu(_mm(cat, wn[...]) + bn[...])
    pn[...] = _mm(jax.nn.relu(_mm(nn, wnc1[...]) + bnc1[...]), wnc2[...]) + bnc2[...]
    pc[...] = _mm(jax.nn.relu(_mm(nn, wc1[...]) + bc1[...]), wc2[...]) + bc2[...]


def _block_call(body, ins, in_widths, out_widths, nrows, r):
    grid = (pl.cdiv(nrows, r),)
    in_specs = []
    for arr, w in zip(ins, in_widths):
        if w is None:
            in_specs.append(_full(arr.shape))
        else:
            in_specs.append(_rows(r, w))
    out_shapes = tuple(jax.ShapeDtypeStruct((nrows, w), jnp.float32)
                       for w in out_widths)
    out_specs = tuple(_rows(r, w) for w in out_widths)
    return pl.pallas_call(
        body, grid=grid, in_specs=in_specs,
        out_specs=out_specs if len(out_widths) > 1 else out_specs[0],
        out_shape=out_shapes if len(out_widths) > 1 else out_shapes[0],
    )(*ins)


def _edge_call(body, ins, in_widths, out_widths):
    return _block_call(body, ins, in_widths, out_widths, E, ER)


def _node_call(body, ins, in_widths, out_widths):
    return _block_call(body, ins, in_widths, out_widths, N, NR)


def kernel(x, edge_attr, edge_index, Wne, bne, Wee, bee, W1e1, b1e1, W1e2,
           b1e2, W1n, b1n, W2e1, b2e1, W2e2, b2e2, W2n, b2n, Weo, beo, Wec,
           bec, Wnc1, bnc1, Wnc2, bnc2, Wc1, bc1, Wc2, bc2):
    f32 = jnp.float32
    src_r = edge_index[0].reshape(NW, KC, CH)
    dst_r = edge_index[1].reshape(NW, KC, CH)
    zer = jnp.zeros((ACCR, D), f32)
    b2r = lambda v: v.reshape(1, -1)

    dsti = edge_index[1]
    own = jnp.minimum(dsti // NRANGE, NW - 1)
    order = jnp.argsort(own, stable=True).astype(jnp.int32)
    wsort = own[order]
    counts = jax.ops.segment_sum(jnp.ones((E,), jnp.int32), own, num_segments=NW)
    starts = jnp.concatenate([jnp.zeros((1,), jnp.int32),
                              jnp.cumsum(counts)[:-1].astype(jnp.int32)])
    pos = jnp.arange(E, dtype=jnp.int32) - starts[wsort]
    flat = wsort * CAPF + pos
    eids_r = (jnp.zeros((NW * CAPF,), jnp.int32).at[flat].set(order)
              .reshape(NW, CAPC, CH))
    ldst_r = (jnp.full((NW * CAPF,), DUMP, jnp.int32)
              .at[flat].set(dsti[order] - wsort * NRANGE)
              .reshape(NW, CAPC, CH))

    nf0 = _node_call(_k_nf0, (x, Wne, b2r(bne)), (D, None, None), (D,))
    ef0 = _edge_call(_k_ef0, (edge_attr, Wee, b2r(bee)), (16, None, None), (D,))

    def scat(e):
        return _sc_scatter(e, eids_r, ldst_r, zer)

    gs, gd = _sc_gather(nf0, src_r, dst_r)
    gs, gd = gs.reshape(E, D), gd.reshape(E, D)
    ef1 = _edge_call(
        _k_edge1, (ef0, gs, gd, W1e1, b2r(b1e1), W1e2, b2r(b1e2)),
        (D, D, D, None, None, None, None), (D,))
    p = scat(ef1)
    nf1 = _node_call(_k_node, (nf0, p, W1n, b2r(b1n)),
                     (D, D, None, None), (D,))

    gs, gd = _sc_gather(nf1, src_r, dst_r)
    gs, gd = gs.reshape(E, D), gd.reshape(E, D)
    ef2, pe = _edge_call(
        _k_edge2,
        (ef1, gs, gd, W1e1, b2r(b1e1), W1e2, b2r(b1e2), Weo, b2r(beo), Wec,
         b2r(bec)),
        (D, D, D, None, None, None, None, None, None, None, None), (D, 1))
    p = scat(ef2)
    nf2 = _node_call(_k_node, (nf1, p, W1n, b2r(b1n)),
                     (D, D, None, None), (D,))

    gs, gd = _sc_gather(nf2, src_r, dst_r)
    gs, gd = gs.reshape(E, D), gd.reshape(E, D)
    efx, efy = _edge_call(
        _k_edge3,
        (ef2, gs, gd, Weo, b2r(beo), W2e1, b2r(b2e1), W2e2, b2r(b2e2)),
        (D, D, D, None, None, None, None, None, None), (D, D))
    px = scat(efx)
    py = scat(efy)
    nf3 = _node_call(_k_node2, (nf2, px, py, W2n, b2r(b2n)),
                     (D, D, D, None, None), (D,))

    gs, gd = _sc_gather(nf3, src_r, dst_r)
    gs, gd = gs.reshape(E, D), gd.reshape(E, D)
    ox, oy = _edge_call(
        _k_edge4,
        (efx, efy, gs, gd, W2e1, b2r(b2e1), W2e2, b2r(b2e2)),
        (D, D, D, D, None, None, None, None), (D, D))
    px = scat(ox)
    py = scat(oy)
    pn, pc = _node_call(
        _k_node_final,
        (nf3, px, py, W2n, b2r(b2n), Wnc1, b2r(bnc1),
         Wnc2, b2r(bnc2), Wc1, b2r(bc1), Wc2, b2r(bc2)),
        (D, D, D, None, None, None, None, None, None, None, None,
         None, None), (1, 8))

    return (pe.reshape(E), pn.reshape(N), pc.reshape(N, 8))


# speedup vs baseline: 1.0044x; 1.0044x over previous
import functools

import jax
import jax.numpy as jnp
from jax import lax
from jax.experimental import pallas as pl
from jax.experimental.pallas import tpu as pltpu
from jax.experimental.pallas import tpu_sc as plsc

N = 10000
E = 320000
D = 128

NC = 2
NS = 16
NW = NC * NS
EPW = E // NW
CH = 80
KC = EPW // CH
RPS = 624
TAIL = N - NS * RPS

ER = 512
NR = 512


def _vs_mesh():
    return plsc.VectorSubcoreMesh(core_axis_name="c", subcore_axis_name="s",
                                  num_cores=NC, num_subcores=NS)


@functools.partial(
    pl.kernel,
    mesh=_vs_mesh(),
    out_type=(
        jax.ShapeDtypeStruct((NW, KC, CH, D), jnp.float32),
        jax.ShapeDtypeStruct((NW, KC, CH, D), jnp.float32),
    ),
    scratch_types=[
        pltpu.VMEM((KC, CH), jnp.int32),
        pltpu.VMEM((KC, CH), jnp.int32),
        pltpu.VMEM((2, CH, D), jnp.float32),
        pltpu.VMEM((2, CH, D), jnp.float32),
        pltpu.SemaphoreType.DMA,
        pltpu.SemaphoreType.DMA,
        pltpu.SemaphoreType.DMA,
        pltpu.SemaphoreType.DMA,
    ],
)
def _sc_gather(nf_hbm, src_hbm, dst_hbm, gs_hbm, gd_hbm,
               idxs, idxd, ab, bb, sa, sb, swa, swb):
    wid = lax.axis_index("s") * NC + lax.axis_index("c")
    pltpu.sync_copy(src_hbm.at[wid], idxs)
    pltpu.sync_copy(dst_hbm.at[wid], idxd)

    def abuf(c):
        return ab.at[lax.rem(c, 2)]

    def bbuf(c):
        return bb.at[lax.rem(c, 2)]

    def issue(c):
        pltpu.async_copy(nf_hbm.at[idxs.at[c]], abuf(c), sa)
        pltpu.async_copy(nf_hbm.at[idxd.at[c]], bbuf(c), sb)

    issue(0)

    @pl.loop(0, KC)
    def _(c):
        pltpu.make_async_copy(nf_hbm.at[idxs.at[c]], abuf(c), sa).wait()
        pltpu.make_async_copy(nf_hbm.at[idxd.at[c]], bbuf(c), sb).wait()

        @pl.when(c >= 1)
        def _():
            pltpu.make_async_copy(abuf(c + 1), gs_hbm.at[wid, c - 1], swa).wait()
            pltpu.make_async_copy(bbuf(c + 1), gd_hbm.at[wid, c - 1], swb).wait()

        @pl.when(c + 1 < KC)
        def _():
            issue(c + 1)

        pltpu.async_copy(abuf(c), gs_hbm.at[wid, c], swa)
        pltpu.async_copy(bbuf(c), gd_hbm.at[wid, c], swb)

    pltpu.make_async_copy(abuf(KC - 1), gs_hbm.at[wid, KC - 1], swa).wait()
    pltpu.make_async_copy(bbuf(KC - 1), gd_hbm.at[wid, KC - 1], swb).wait()


NRANGE = 312
ACCR = 336
DUMP = 328
SCC = 128
CAPC = 98
CAPF = CAPC * SCC


def _make_scatter(w):
    @functools.partial(
        pl.kernel,
        mesh=_vs_mesh(),
        out_type=jax.ShapeDtypeStruct((N, w), jnp.float32),
        scratch_types=[
            pltpu.VMEM_SHARED((NS, ACCR, w), jnp.float32),
            pltpu.VMEM((CAPC, SCC), jnp.int32),
            pltpu.VMEM((CAPC, SCC), jnp.int32),
            pltpu.VMEM((2, SCC, w), jnp.float32),
            pltpu.SemaphoreType.DMA,
        ],
    )
    def _scat(ef_hbm, eids_hbm, ldst_hbm, zer_hbm, out_hbm,
              acc, eids, ldst, vb, sem):
        cid = lax.axis_index("c")
        sid = lax.axis_index("s")
        wid = sid * NC + cid
        pltpu.sync_copy(zer_hbm, acc.at[sid])
        pltpu.sync_copy(eids_hbm.at[wid], eids)
        pltpu.sync_copy(ldst_hbm.at[wid], ldst)

        def vbuf(c):
            return vb.at[lax.rem(c, 2)]

        pltpu.async_copy(ef_hbm.at[eids.at[0]], vbuf(0), sem)

        @pl.loop(0, CAPC)
        def _(c):
            pltpu.make_async_copy(ef_hbm.at[eids.at[c]], vbuf(c), sem).wait()

            @pl.when(c + 1 < CAPC)
            def _():
                pltpu.async_copy(ef_hbm.at[eids.at[c + 1]], vbuf(c + 1), sem)

            pltpu.sync_copy(vbuf(c), acc.at[sid].at[ldst.at[c]], add=True)

        pltpu.sync_copy(acc.at[sid].at[pl.ds(0, NRANGE)],
                        out_hbm.at[pl.ds(wid * NRANGE, NRANGE)])

        @pl.when(wid == NW - 1)
        def _():
            pltpu.sync_copy(acc.at[sid].at[pl.ds(NRANGE, TAIL)],
                            out_hbm.at[pl.ds(NW * NRANGE, TAIL)])

    return _scat


_sc_scatter = _make_scatter(D)


def _full(shape):
    return pl.BlockSpec(shape, lambda i: (0,) * len(shape))


def _rows(r, width):
    return pl.BlockSpec((r, width), lambda i: (i, 0))


def _mm(a, b):
    return jnp.dot(a, b, preferred_element_type=jnp.float32)


def _k_nf0(x, wne, bne, nf):
    nf[...] = _mm(x[...], wne[...]) + bne[...]


def _k_ef0(ea, wee, bee, ef):
    ef[...] = _mm(ea[...], wee[...]) + bee[...]


def _k_edge1(ef0, gs, gd, w1, b1, w2, b2, ef1):
    e_in = jnp.concatenate([gs[...], gd[...], ef0[...]], axis=1)
    h = jax.nn.relu(_mm(e_in, w1[...]) + b1[...])
    ef1[...] = jax.nn.relu(_mm(h, w2[...]) + b2[...])


def _k_edge2(ef1, gs, gd, w1, b1, w2, b2, weo, beo, wec, bec, ef2, pe):
    e_in = jnp.concatenate([gs[...], gd[...], ef1[...]], axis=1)
    h = jax.nn.relu(_mm(e_in, w1[...]) + b1[...])
    e2 = jax.nn.relu(_mm(h, w2[...]) + b2[...])
    ef2[...] = e2
    eo = jax.nn.relu(_mm(e2, weo[...]) + beo[...])
    pe[...] = _mm(eo, wec[...]) + bec[...]


def _k_edge3(ef2, gs, gd, weo, beo, w21, b21, w22, b22, efx, efy):
    e2 = ef2[...]
    eo = jax.nn.relu(_mm(e2, weo[...]) + beo[...])
    e_in = jnp.concatenate([gs[...], gd[...], eo, e2], axis=1)
    h = jax.nn.relu(_mm(e_in, w21[...]) + b21[...])
    e3 = jax.nn.relu(_mm(h, w22[...]) + b22[...])
    efx[...] = e3[:, :128]
    efy[...] = e3[:, 32:]


def _k_edge4(efx, efy, gs, gd, w21, b21, w22, b22, ox, oy):
    e_in = jnp.concatenate([gs[...], gd[...], efx[...], efy[...][:, 96:]],
                           axis=1)
    h = jax.nn.relu(_mm(e_in, w21[...]) + b21[...])
    e4 = jax.nn.relu(_mm(h, w22[...]) + b22[...])
    ox[...] = e4[:, :128]
    oy[...] = e4[:, 32:]


def _k_node(nf, p, wn, bn, nfo):
    cat = jnp.concatenate([nf[...], p[...]], axis=1)
    nfo[...] = jax.nn.relu(_mm(cat, wn[...]) + bn[...])


def _node2_cat(nf, px, py):
    return jnp.concatenate([nf[...], px[...], py[...][:, 96:]], axis=1)


def _k_node2(nf, px, py, wn, bn, nfo):
    cat = _node2_cat(nf, px, py)
    nfo[...] = jax.nn.relu(_mm(cat, wn[...]) + bn[...])


def _k_node_final(nf, px, py, wn, bn, wnc1, bnc1, wnc2, bnc2,
                  wc1, bc1, wc2, bc2, pn, pc):
    cat = _node2_cat(nf, px, py)
    nn = jax.nn.relu(_mm(cat, wn[...]) + bn[...])
    pn[...] = _mm(jax.nn.relu(_mm(nn, wnc1[...]) + bnc1[...]), wnc2[...]) + bnc2[...]
    pc[...] = _mm(jax.nn.relu(_mm(nn, wc1[...]) + bc1[...]), wc2[...]) + bc2[...]


def _block_call(body, ins, in_widths, out_widths, nrows, r):
    grid = (pl.cdiv(nrows, r),)
    in_specs = []
    for arr, w in zip(ins, in_widths):
        if w is None:
            in_specs.append(_full(arr.shape))
        else:
            in_specs.append(_rows(r, w))
    out_shapes = tuple(jax.ShapeDtypeStruct((nrows, w), jnp.float32)
                       for w in out_widths)
    out_specs = tuple(_rows(r, w) for w in out_widths)
    return pl.pallas_call(
        body, grid=grid, in_specs=in_specs,
        out_specs=out_specs if len(out_widths) > 1 else out_specs[0],
        out_shape=out_shapes if len(out_widths) > 1 else out_shapes[0],
    )(*ins)


def _edge_call(body, ins, in_widths, out_widths):
    return _block_call(body, ins, in_widths, out_widths, E, ER)


def _node_call(body, ins, in_widths, out_widths):
    return _block_call(body, ins, in_widths, out_widths, N, NR)


def kernel(x, edge_attr, edge_index, Wne, bne, Wee, bee, W1e1, b1e1, W1e2,
           b1e2, W1n, b1n, W2e1, b2e1, W2e2, b2e2, W2n, b2n, Weo, beo, Wec,
           bec, Wnc1, bnc1, Wnc2, bnc2, Wc1, bc1, Wc2, bc2):
    f32 = jnp.float32
    src_r = edge_index[0].reshape(NW, KC, CH)
    dst_r = edge_index[1].reshape(NW, KC, CH)
    zer = jnp.zeros((ACCR, D), f32)
    b2r = lambda v: v.reshape(1, -1)

    dsti = edge_index[1]
    own = jnp.minimum(dsti // NRANGE, NW - 1)
    order = jnp.argsort(own, stable=True).astype(jnp.int32)
    wsort = own[order]
    counts = jax.ops.segment_sum(jnp.ones((E,), jnp.int32), own, num_segments=NW)
    starts = jnp.concatenate([jnp.zeros((1,), jnp.int32),
                              jnp.cumsum(counts)[:-1].astype(jnp.int32)])
    pos = jnp.arange(E, dtype=jnp.int32) - starts[wsort]
    flat = wsort * CAPF + pos
    eids_r = (jnp.zeros((NW * CAPF,), jnp.int32).at[flat].set(order)
              .reshape(NW, CAPC, SCC))
    ldst_r = (jnp.full((NW * CAPF,), DUMP, jnp.int32)
              .at[flat].set(dsti[order] - wsort * NRANGE)
              .reshape(NW, CAPC, SCC))

    nf0 = _node_call(_k_nf0, (x, Wne, b2r(bne)), (D, None, None), (D,))
    ef0 = _edge_call(_k_ef0, (edge_attr, Wee, b2r(bee)), (16, None, None), (D,))

    def scat(e):
        return _sc_scatter(e, eids_r, ldst_r, zer)

    gs, gd = _sc_gather(nf0, src_r, dst_r)
    gs, gd = gs.reshape(E, D), gd.reshape(E, D)
    ef1 = _edge_call(
        _k_edge1, (ef0, gs, gd, W1e1, b2r(b1e1), W1e2, b2r(b1e2)),
        (D, D, D, None, None, None, None), (D,))
    p = scat(ef1)
    nf1 = _node_call(_k_node, (nf0, p, W1n, b2r(b1n)),
                     (D, D, None, None), (D,))

    gs, gd = _sc_gather(nf1, src_r, dst_r)
    gs, gd = gs.reshape(E, D), gd.reshape(E, D)
    ef2, pe = _edge_call(
        _k_edge2,
        (ef1, gs, gd, W1e1, b2r(b1e1), W1e2, b2r(b1e2), Weo, b2r(beo), Wec,
         b2r(bec)),
        (D, D, D, None, None, None, None, None, None, None, None), (D, 1))
    p = scat(ef2)
    nf2 = _node_call(_k_node, (nf1, p, W1n, b2r(b1n)),
                     (D, D, None, None), (D,))

    gs, gd = _sc_gather(nf2, src_r, dst_r)
    gs, gd = gs.reshape(E, D), gd.reshape(E, D)
    efx, efy = _edge_call(
        _k_edge3,
        (ef2, gs, gd, Weo, b2r(beo), W2e1, b2r(b2e1), W2e2, b2r(b2e2)),
        (D, D, D, None, None, None, None, None, None), (D, D))
    px = scat(efx)
    py = scat(efy)
    nf3 = _node_call(_k_node2, (nf2, px, py, W2n, b2r(b2n)),
                     (D, D, D, None, None), (D,))

    gs, gd = _sc_gather(nf3, src_r, dst_r)
    gs, gd = gs.reshape(E, D), gd.reshape(E, D)
    ox, oy = _edge_call(
        _k_edge4,
        (efx, efy, gs, gd, W2e1, b2r(b2e1), W2e2, b2r(b2e2)),
        (D, D, D, D, None, None, None, None), (D, D))
    px = scat(ox)
    py = scat(oy)
    pn, pc = _node_call(
        _k_node_final,
        (nf3, px, py, W2n, b2r(b2n), Wnc1, b2r(bnc1),
         Wnc2, b2r(bnc2), Wc1, b2r(bc1), Wc2, b2r(bc2)),
        (D, D, D, None, None, None, None, None, None, None, None,
         None, None), (1, 8))

    return (pe.reshape(E), pn.reshape(N), pc.reshape(N, 8))
